# baseline (speedup 1.0000x reference)
import jax
import jax.numpy as jnp
from jax.experimental import pallas as pl
from jax.experimental.pallas import tpu as pltpu

_N = 100000
_D = 128


def _body(c_ref, rgbf, flowf, rgb_in, flow_in, counts,
          rgb_out, flow_out,
          row_rgb, row_flow, n_ref,
          sem_b1, sem_b2, sem_r1, sem_r2, sem_n):
    c = c_ref[0]
    cp1 = pltpu.make_async_copy(rgb_in, rgb_out, sem_b1)
    cp2 = pltpu.make_async_copy(flow_in, flow_out, sem_b2)
    cp1.start()
    cp2.start()
    base = pl.multiple_of((c // 128) * 128, 128)
    gn = pltpu.make_async_copy(counts.at[pl.ds(base, 128)], n_ref, sem_n)
    g1 = pltpu.make_async_copy(rgb_in.at[pl.ds(c, 1)], row_rgb, sem_r1)
    g2 = pltpu.make_async_copy(flow_in.at[pl.ds(c, 1)], row_flow, sem_r2)
    gn.start()
    g1.start()
    g2.start()
    gn.wait()
    g1.wait()
    g2.wait()
    n = n_ref[c - base]
    scale = n / (n + 1.0)
    inv = 1.0 / (n + 1.0)
    row_rgb[...] = scale * row_rgb[...] + inv * rgbf[...]
    row_flow[...] = scale * row_flow[...] + inv * flowf[...]
    cp1.wait()
    cp2.wait()
    s1 = pltpu.make_async_copy(row_rgb, rgb_out.at[pl.ds(c, 1)], sem_r1)
    s2 = pltpu.make_async_copy(row_flow, flow_out.at[pl.ds(c, 1)], sem_r2)
    s1.start()
    s2.start()
    s1.wait()
    s2.wait()


def kernel(rgb_feat, flow_feat, pseudo_label, rgb_prototypes, flow_prototypes, counts):
    c = jnp.asarray(pseudo_label, jnp.int32).reshape(1)
    rgb_f = rgb_feat.reshape(1, _D)
    flow_f = flow_feat.reshape(1, _D)
    out = pl.pallas_call(
        _body,
        in_specs=[
            pl.BlockSpec(memory_space=pltpu.SMEM),
            pl.BlockSpec(memory_space=pltpu.VMEM),
            pl.BlockSpec(memory_space=pltpu.VMEM),
            pl.BlockSpec(memory_space=pl.ANY),
            pl.BlockSpec(memory_space=pl.ANY),
            pl.BlockSpec(memory_space=pl.ANY),
        ],
        out_specs=[
            pl.BlockSpec(memory_space=pl.ANY),
            pl.BlockSpec(memory_space=pl.ANY),
        ],
        out_shape=[
            jax.ShapeDtypeStruct((_N, _D), jnp.float32),
            jax.ShapeDtypeStruct((_N, _D), jnp.float32),
        ],
        scratch_shapes=[
            pltpu.VMEM((1, _D), jnp.float32),
            pltpu.VMEM((1, _D), jnp.float32),
            pltpu.SMEM((128,), jnp.float32),
            pltpu.SemaphoreType.DMA,
            pltpu.SemaphoreType.DMA,
            pltpu.SemaphoreType.DMA,
            pltpu.SemaphoreType.DMA,
            pltpu.SemaphoreType.DMA,
        ],
    )(c, rgb_f, flow_f, rgb_prototypes, flow_prototypes, counts)
    return (out[0], out[1])


# speedup vs baseline: 25.8265x; 25.8265x over previous
import jax
import jax.numpy as jnp
from jax.experimental import pallas as pl
from jax.experimental.pallas import tpu as pltpu

_N = 100000
_D = 128
_BR = 5000


def _body(c_ref, rgb_f_ref, flow_f_ref, rgb_in, flow_in, counts_ref,
          rgb_out, flow_out):
    i = pl.program_id(0)
    c = c_ref[0]
    rgb_out[...] = rgb_in[...]
    flow_out[...] = flow_in[...]

    @pl.when(i == c // _BR)
    def _blend():
        rows = i * _BR + jax.lax.broadcasted_iota(jnp.int32, (_BR, 1), 0)
        mask = rows == c
        n = counts_ref[...]
        scale = n / (n + 1.0)
        inv = 1.0 / (n + 1.0)
        rgb_out[...] = jnp.where(
            mask, scale * rgb_in[...] + inv * rgb_f_ref[...], rgb_in[...])
        flow_out[...] = jnp.where(
            mask, scale * flow_in[...] + inv * flow_f_ref[...], flow_in[...])


def kernel(rgb_feat, flow_feat, pseudo_label, rgb_prototypes, flow_prototypes, counts):
    c = jnp.asarray(pseudo_label, jnp.int32).reshape(1)
    rgb_f = rgb_feat.reshape(1, _D)
    flow_f = flow_feat.reshape(1, _D)
    counts2 = counts.reshape(_N, 1)
    grid = _N // _BR
    out = pl.pallas_call(
        _body,
        grid=(grid,),
        in_specs=[
            pl.BlockSpec(memory_space=pltpu.SMEM),
            pl.BlockSpec((1, _D), lambda i: (0, 0)),
            pl.BlockSpec((1, _D), lambda i: (0, 0)),
            pl.BlockSpec((_BR, _D), lambda i: (i, 0)),
            pl.BlockSpec((_BR, _D), lambda i: (i, 0)),
            pl.BlockSpec((_BR, 1), lambda i: (i, 0)),
        ],
        out_specs=[
            pl.BlockSpec((_BR, _D), lambda i: (i, 0)),
            pl.BlockSpec((_BR, _D), lambda i: (i, 0)),
        ],
        out_shape=[
            jax.ShapeDtypeStruct((_N, _D), jnp.float32),
            jax.ShapeDtypeStruct((_N, _D), jnp.float32),
        ],
        compiler_params=pltpu.CompilerParams(
            dimension_semantics=("arbitrary",),
        ),
    )(c, rgb_f, flow_f, rgb_prototypes, flow_prototypes, counts2)
    return (out[0], out[1])


# speedup vs baseline: 45.5977x; 1.7655x over previous
import jax
import jax.numpy as jnp
from jax.experimental import pallas as pl
from jax.experimental.pallas import tpu as pltpu

_N = 100000
_D = 128
_BR = 5000


def _body(c_ref, rgb_f_ref, flow_f_ref, rgb_in, flow_in, counts,
          rgb_out, flow_out, n_ref, sem_n):
    i = pl.program_id(0)
    c = c_ref[0]
    rgb_out[...] = rgb_in[...]
    flow_out[...] = flow_in[...]

    @pl.when(i == c // _BR)
    def _blend():
        base = pl.multiple_of((c // 128) * 128, 128)
        gn = pltpu.make_async_copy(counts.at[pl.ds(base, 128)], n_ref, sem_n)
        gn.start()
        gn.wait()
        n = n_ref[c - base]
        scale = n / (n + 1.0)
        inv = 1.0 / (n + 1.0)
        rows = i * _BR + jax.lax.broadcasted_iota(jnp.int32, (_BR, 1), 0)
        mask = rows == c
        rgb_out[...] = jnp.where(
            mask, scale * rgb_in[...] + inv * rgb_f_ref[...], rgb_in[...])
        flow_out[...] = jnp.where(
            mask, scale * flow_in[...] + inv * flow_f_ref[...], flow_in[...])


def kernel(rgb_feat, flow_feat, pseudo_label, rgb_prototypes, flow_prototypes, counts):
    c = jnp.asarray(pseudo_label, jnp.int32).reshape(1)
    rgb_f = rgb_feat.reshape(1, _D)
    flow_f = flow_feat.reshape(1, _D)
    grid = _N // _BR
    out = pl.pallas_call(
        _body,
        grid=(grid,),
        in_specs=[
            pl.BlockSpec(memory_space=pltpu.SMEM),
            pl.BlockSpec((1, _D), lambda i: (0, 0)),
            pl.BlockSpec((1, _D), lambda i: (0, 0)),
            pl.BlockSpec((_BR, _D), lambda i: (i, 0)),
            pl.BlockSpec((_BR, _D), lambda i: (i, 0)),
            pl.BlockSpec(memory_space=pl.ANY),
        ],
        out_specs=[
            pl.BlockSpec((_BR, _D), lambda i: (i, 0)),
            pl.BlockSpec((_BR, _D), lambda i: (i, 0)),
        ],
        out_shape=[
            jax.ShapeDtypeStruct((_N, _D), jnp.float32),
            jax.ShapeDtypeStruct((_N, _D), jnp.float32),
        ],
        scratch_shapes=[
            pltpu.SMEM((128,), jnp.float32),
            pltpu.SemaphoreType.DMA,
        ],
        compiler_params=pltpu.CompilerParams(
            dimension_semantics=("arbitrary",),
        ),
    )(c, rgb_f, flow_f, rgb_prototypes, flow_prototypes, counts)
    return (out[0], out[1])
